# baseline (speedup 1.0000x reference)
import functools
import math

import jax
import jax.numpy as jnp
from jax import lax
from jax.experimental import pallas as pl
from jax.experimental.pallas import tpu as pltpu
from jax.experimental.pallas import tpu_sc as plsc

NUM_SAMPLES = 100000
NUM_FEATURES = 128
BATCH = 1024
TEMP = 0.02
LOG2E = math.log2(math.e)

NSTREAM = 10
BLOCK_N = 1000
NB = NUM_SAMPLES // (NSTREAM * BLOCK_N)


def _prep_body(img_ref, img2_ref):
    img = img_ref[...]
    n = jnp.sqrt(jnp.sum(img * img, axis=1, keepdims=True))
    img2_ref[...] = img * (LOG2E / TEMP / jnp.maximum(n, 1e-12))


def _stream_body(img2_ref, *refs):
    f_refs, acc_ref, acc_scr = refs[:NSTREAM], refs[NSTREAM], refs[NSTREAM + 1]
    step = pl.program_id(0)

    @pl.when(step == 0)
    def _init():
        acc_scr[...] = jnp.zeros_like(acc_scr)

    img2 = img2_ref[...]
    s = jnp.zeros((BATCH, 1), jnp.float32)
    for f_ref in f_refs:
        z = lax.dot_general(
            img2, f_ref[...], (((1,), (1,)), ((), ())),
            preferred_element_type=jnp.float32)
        s = s + jnp.sum(jnp.exp2(z), axis=1, keepdims=True)
    acc_scr[...] += s

    @pl.when(step == NB - 1)
    def _fin():
        acc_ref[...] = acc_scr[...]


def _combine_body(img2_ref, g_ref, acc_ref, out_ref):
    tgt = jnp.sum(img2_ref[...] * g_ref[...], axis=1, keepdims=True)
    lse2 = jnp.log(acc_ref[...]) * LOG2E
    out_ref[...] = (jnp.mean(lse2 - tgt) * (1.0 / LOG2E)).reshape(1, 1)


def _tc_loss(img, feats, tgt_idx):
    img2 = pl.pallas_call(
        _prep_body,
        out_shape=jax.ShapeDtypeStruct((BATCH, NUM_FEATURES), jnp.float32),
    )(img)

    def _feat_spec(s):
        return pl.BlockSpec((BLOCK_N, NUM_FEATURES), lambda i, s=s: (s * NB + i, 0))

    acc = pl.pallas_call(
        _stream_body,
        grid=(NB,),
        in_specs=[pl.BlockSpec((BATCH, NUM_FEATURES), lambda i: (0, 0))]
        + [_feat_spec(s) for s in range(NSTREAM)],
        out_specs=pl.BlockSpec((BATCH, 1), lambda i: (0, 0)),
        out_shape=jax.ShapeDtypeStruct((BATCH, 1), jnp.float32),
        scratch_shapes=[pltpu.VMEM((BATCH, 1), jnp.float32)],
        compiler_params=pltpu.CompilerParams(
            dimension_semantics=("arbitrary",)),
    )(img2, *([feats] * NSTREAM))

    g = _make_sc_gather()(feats, tgt_idx)
    out = pl.pallas_call(
        _combine_body,
        out_shape=jax.ShapeDtypeStruct((1, 1), jnp.float32),
    )(img2, g, acc)
    return out[0, 0]


def _make_sc_gather():
    info = plsc.get_sparse_core_info()
    nc, ns = 1, info.num_subcores
    nw = nc * ns
    b_per_w = BATCH // nw
    mesh = plsc.VectorSubcoreMesh(
        core_axis_name="c", subcore_axis_name="s", num_cores=nc)

    @functools.partial(
        pl.kernel, mesh=mesh,
        out_type=jax.ShapeDtypeStruct((BATCH, NUM_FEATURES), jnp.float32),
        scratch_types=[
            pltpu.VMEM((b_per_w,), jnp.int32),
            pltpu.VMEM((b_per_w, NUM_FEATURES), jnp.float32),
            pltpu.SemaphoreType.DMA,
        ],
    )
    def sc_gather(table_hbm, idx_hbm, out_hbm, idx_v, rows_v, sem):
        wid = lax.axis_index("s") * nc + lax.axis_index("c")
        base = wid * b_per_w
        pltpu.sync_copy(idx_hbm.at[pl.ds(base, b_per_w)], idx_v)
        pltpu.async_copy(table_hbm.at[idx_v], rows_v, sem).wait()
        pltpu.sync_copy(rows_v, out_hbm.at[pl.ds(base, b_per_w)])

    return sc_gather


def kernel(image_inputs, text_inputs, targets, features):
    del text_inputs
    return _tc_loss(image_inputs, features, targets.astype(jnp.int32))


# speedup vs baseline: 1.0291x; 1.0291x over previous
import functools
import math

import jax
import jax.numpy as jnp
from jax import lax
from jax.experimental import pallas as pl
from jax.experimental.pallas import tpu as pltpu
from jax.experimental.pallas import tpu_sc as plsc

NUM_SAMPLES = 100000
NUM_FEATURES = 128
BATCH = 1024
TEMP = 0.02
LOG2E = math.log2(math.e)

NSTREAM = 5
BLOCK_N = 2000
NB = NUM_SAMPLES // (NSTREAM * BLOCK_N)


def _prep_body(img_ref, img2_ref):
    img = img_ref[...]
    n = jnp.sqrt(jnp.sum(img * img, axis=1, keepdims=True))
    img2_ref[...] = img * (LOG2E / TEMP / jnp.maximum(n, 1e-12))


def _stream_body(img2_ref, f0_ref, f1_ref, f2_ref, f3_ref, f4_ref, acc_ref, acc_scr):
    step = pl.program_id(0)

    @pl.when(step == 0)
    def _init():
        acc_scr[...] = jnp.zeros_like(acc_scr)

    img2 = img2_ref[...]
    s = jnp.zeros((BATCH, 1), jnp.float32)
    for f_ref in (f0_ref, f1_ref, f2_ref, f3_ref, f4_ref):
        z = lax.dot_general(
            img2, f_ref[...], (((1,), (1,)), ((), ())),
            preferred_element_type=jnp.float32)
        s = s + jnp.sum(jnp.exp2(z), axis=1, keepdims=True)
    acc_scr[...] += s

    @pl.when(step == NB - 1)
    def _fin():
        acc_ref[...] = acc_scr[...]


def _combine_body(img2_ref, g_ref, acc_ref, out_ref):
    tgt = jnp.sum(img2_ref[...] * g_ref[...], axis=1, keepdims=True)
    lse2 = jnp.log(acc_ref[...]) * LOG2E
    out_ref[...] = (jnp.mean(lse2 - tgt) * (1.0 / LOG2E)).reshape(1, 1)


def _tc_loss(img, feats, tgt_idx):
    img2 = pl.pallas_call(
        _prep_body,
        out_shape=jax.ShapeDtypeStruct((BATCH, NUM_FEATURES), jnp.float32),
    )(img)

    def _feat_spec(s):
        return pl.BlockSpec((BLOCK_N, NUM_FEATURES), lambda i, s=s: (s * NB + i, 0))

    acc = pl.pallas_call(
        _stream_body,
        grid=(NB,),
        in_specs=[pl.BlockSpec((BATCH, NUM_FEATURES), lambda i: (0, 0))]
        + [_feat_spec(s) for s in range(NSTREAM)],
        out_specs=pl.BlockSpec((BATCH, 1), lambda i: (0, 0)),
        out_shape=jax.ShapeDtypeStruct((BATCH, 1), jnp.float32),
        scratch_shapes=[pltpu.VMEM((BATCH, 1), jnp.float32)],
        compiler_params=pltpu.CompilerParams(
            dimension_semantics=("arbitrary",)),
    )(img2, feats, feats, feats, feats, feats)

    g = _make_sc_gather()(feats, tgt_idx)
    out = pl.pallas_call(
        _combine_body,
        out_shape=jax.ShapeDtypeStruct((1, 1), jnp.float32),
    )(img2, g, acc)
    return out[0, 0]


def _make_sc_gather():
    info = plsc.get_sparse_core_info()
    nc, ns = 1, info.num_subcores
    nw = nc * ns
    b_per_w = BATCH // nw
    mesh = plsc.VectorSubcoreMesh(
        core_axis_name="c", subcore_axis_name="s", num_cores=nc)

    @functools.partial(
        pl.kernel, mesh=mesh,
        out_type=jax.ShapeDtypeStruct((BATCH, NUM_FEATURES), jnp.float32),
        scratch_types=[
            pltpu.VMEM((b_per_w,), jnp.int32),
            pltpu.VMEM((b_per_w, NUM_FEATURES), jnp.float32),
            pltpu.SemaphoreType.DMA,
        ],
    )
    def sc_gather(table_hbm, idx_hbm, out_hbm, idx_v, rows_v, sem):
        wid = lax.axis_index("s") * nc + lax.axis_index("c")
        base = wid * b_per_w
        pltpu.sync_copy(idx_hbm.at[pl.ds(base, b_per_w)], idx_v)
        pltpu.async_copy(table_hbm.at[idx_v], rows_v, sem).wait()
        pltpu.sync_copy(rows_v, out_hbm.at[pl.ds(base, b_per_w)])

    return sc_gather


def kernel(image_inputs, text_inputs, targets, features):
    del text_inputs
    return _tc_loss(image_inputs, features, targets.astype(jnp.int32))


# speedup vs baseline: 1.0361x; 1.0068x over previous
import functools
import math

import jax
import jax.numpy as jnp
from jax import lax
from jax.experimental import pallas as pl
from jax.experimental.pallas import tpu as pltpu
from jax.experimental.pallas import tpu_sc as plsc

NUM_SAMPLES = 100000
NUM_FEATURES = 128
BATCH = 1024
TEMP = 0.02
LOG2E = math.log2(math.e)

NSTREAM = 5
BLOCK_N = 4000
NB = NUM_SAMPLES // (NSTREAM * BLOCK_N)


def _prep_body(img_ref, img2_ref):
    img = img_ref[...]
    n = jnp.sqrt(jnp.sum(img * img, axis=1, keepdims=True))
    img2_ref[...] = img * (LOG2E / TEMP / jnp.maximum(n, 1e-12))


def _stream_body(img2_ref, f0_ref, f1_ref, f2_ref, f3_ref, f4_ref, acc_ref, acc_scr):
    step = pl.program_id(0)

    @pl.when(step == 0)
    def _init():
        acc_scr[...] = jnp.zeros_like(acc_scr)

    img2 = img2_ref[...]
    s = jnp.zeros((BATCH, 1), jnp.float32)
    for f_ref in (f0_ref, f1_ref, f2_ref, f3_ref, f4_ref):
        z = lax.dot_general(
            img2, f_ref[...], (((1,), (1,)), ((), ())),
            preferred_element_type=jnp.float32)
        s = s + jnp.sum(jnp.exp2(z), axis=1, keepdims=True)
    acc_scr[...] += s

    @pl.when(step == NB - 1)
    def _fin():
        acc_ref[...] = acc_scr[...]


def _combine_body(img2_ref, g_ref, acc_ref, out_ref):
    tgt = jnp.sum(img2_ref[...] * g_ref[...], axis=1, keepdims=True)
    lse2 = jnp.log(acc_ref[...]) * LOG2E
    out_ref[...] = (jnp.mean(lse2 - tgt) * (1.0 / LOG2E)).reshape(1, 1)


def _tc_loss(img, feats, tgt_idx):
    img2 = pl.pallas_call(
        _prep_body,
        out_shape=jax.ShapeDtypeStruct((BATCH, NUM_FEATURES), jnp.float32),
    )(img)

    def _feat_spec(s):
        return pl.BlockSpec((BLOCK_N, NUM_FEATURES), lambda i, s=s: (s * NB + i, 0))

    acc = pl.pallas_call(
        _stream_body,
        grid=(NB,),
        in_specs=[pl.BlockSpec((BATCH, NUM_FEATURES), lambda i: (0, 0))]
        + [_feat_spec(s) for s in range(NSTREAM)],
        out_specs=pl.BlockSpec((BATCH, 1), lambda i: (0, 0)),
        out_shape=jax.ShapeDtypeStruct((BATCH, 1), jnp.float32),
        scratch_shapes=[pltpu.VMEM((BATCH, 1), jnp.float32)],
        compiler_params=pltpu.CompilerParams(
            dimension_semantics=("arbitrary",)),
    )(img2, feats, feats, feats, feats, feats)

    g = _make_sc_gather()(feats, tgt_idx)
    out = pl.pallas_call(
        _combine_body,
        out_shape=jax.ShapeDtypeStruct((1, 1), jnp.float32),
    )(img2, g, acc)
    return out[0, 0]


def _make_sc_gather():
    info = plsc.get_sparse_core_info()
    nc, ns = 1, info.num_subcores
    nw = nc * ns
    b_per_w = BATCH // nw
    mesh = plsc.VectorSubcoreMesh(
        core_axis_name="c", subcore_axis_name="s", num_cores=nc)

    @functools.partial(
        pl.kernel, mesh=mesh,
        out_type=jax.ShapeDtypeStruct((BATCH, NUM_FEATURES), jnp.float32),
        scratch_types=[
            pltpu.VMEM((b_per_w,), jnp.int32),
            pltpu.VMEM((b_per_w, NUM_FEATURES), jnp.float32),
            pltpu.SemaphoreType.DMA,
        ],
    )
    def sc_gather(table_hbm, idx_hbm, out_hbm, idx_v, rows_v, sem):
        wid = lax.axis_index("s") * nc + lax.axis_index("c")
        base = wid * b_per_w
        pltpu.sync_copy(idx_hbm.at[pl.ds(base, b_per_w)], idx_v)
        pltpu.async_copy(table_hbm.at[idx_v], rows_v, sem).wait()
        pltpu.sync_copy(rows_v, out_hbm.at[pl.ds(base, b_per_w)])

    return sc_gather


def kernel(image_inputs, text_inputs, targets, features):
    del text_inputs
    return _tc_loss(image_inputs, features, targets.astype(jnp.int32))
